# baseline (speedup 1.0000x reference)
import jax
import jax.numpy as jnp
from jax.experimental import pallas as pl


def _add_kernel(x_ref, pos_ref, o_ref):
    o_ref[...] = x_ref[...] + pos_ref[...]


def kernel(x, pos_emb):
    batch, seq_len, dim = x.shape
    flat = seq_len * dim
    x2 = x.reshape(batch, flat)
    pos = pos_emb[:seq_len].reshape(1, flat)
    blk = 256
    grid = (batch // blk,)
    out = pl.pallas_call(
        _add_kernel,
        grid=grid,
        in_specs=[
            pl.BlockSpec((blk, flat), lambda i: (i, 0)),
            pl.BlockSpec((1, flat), lambda i: (0, 0)),
        ],
        out_specs=pl.BlockSpec((blk, flat), lambda i: (i, 0)),
        out_shape=jax.ShapeDtypeStruct((batch, flat), x.dtype),
    )(x2, pos)
    return out.reshape(batch, seq_len, dim)
